# Initial kernel scaffold; baseline (speedup 1.0000x reference)
import jax
import jax.numpy as jnp
from jax.experimental import pallas as pl


def kernel(user_emb, entity_emb, graph_indices, graph_values):
    raise NotImplementedError("write your pallas kernel here")


# speedup vs baseline: 4.2630x; 4.2630x over previous
import functools

import jax
import jax.numpy as jnp
from jax import lax
from jax.experimental import pallas as pl
from jax.experimental.pallas import tpu as pltpu
from jax.experimental.pallas import tpu_sc as plsc

N_USERS = 2000
N = 10000
NP = 10240
D = 128
E = 320000
N_HOPS = 3

NC = 2
NS = 16
DH = D // NC
RPT = NP // NS
K = 128
NCH = 160
EPT = NCH * K
E_PAD = NS * EPT
NQ = RPT // K


def _splat(i):
    return jnp.full((16,), i, dtype=jnp.int32)


_GDN = lax.GatherDimensionNumbers(
    offset_dims=(), collapsed_slice_dims=(0,), start_index_map=(0,))


def _bcast_lane(v16, lane):
    return lax.gather(v16, _splat(lane)[:, None], _GDN, (1,),
                      mode=lax.GatherScatterMode.PROMISE_IN_BOUNDS)


def _sc_body(emb2, er, wr4, out2, s0, s1, ebuf, wbuf, gbuf, abuf):
    c = lax.axis_index("c")
    s = lax.axis_index("s")
    row0 = s * RPT

    for q in range(NQ):
        sl = pl.ds(row0 + K * q, K)
        pltpu.sync_copy(emb2.at[c, sl], abuf)
        pltpu.sync_copy(abuf, s0.at[sl])
    plsc.subcore_barrier()

    for hop in range(N_HOPS):
        cur = s0 if hop % 2 == 0 else s1
        nxt = s1 if hop % 2 == 0 else s0

        def _zrow(i, _):
            for k in range(DH // 16):
                gbuf[i, pl.ds(16 * k, 16)] = jnp.zeros((16,), jnp.float32)
            return 0

        lax.fori_loop(0, K, _zrow, 0)
        for q in range(NQ):
            pltpu.sync_copy(gbuf, nxt.at[pl.ds(row0 + K * q, K)])
        plsc.subcore_barrier()

        def _chunk(j, _):
            pltpu.sync_copy(er.at[s, j], ebuf)
            pltpu.sync_copy(wr4.at[s, j], wbuf)
            pltpu.sync_copy(cur.at[ebuf.at[0]], gbuf)

            def _scale16(g, _):
                w16 = wbuf[0, pl.ds(16 * g, 16)]
                base = 16 * g
                for e16 in range(16):
                    wbc = _bcast_lane(w16, e16)
                    for k in range(DH // 16):
                        sl = pl.ds(16 * k, 16)
                        gbuf[base + e16, sl] = gbuf[base + e16, sl] * wbc
                return 0

            lax.fori_loop(0, K // 16, _scale16, 0)
            pltpu.sync_copy(gbuf, nxt.at[ebuf.at[1]], add=True)
            return 0

        lax.fori_loop(0, NCH, _chunk, 0)
        plsc.subcore_barrier()

        for q in range(NQ):
            sl = pl.ds(row0 + K * q, K)
            pltpu.sync_copy(nxt.at[sl], gbuf)
            if hop == 0:
                pltpu.sync_copy(emb2.at[c, sl], abuf)
            else:
                pltpu.sync_copy(out2.at[c, sl], abuf)

            def _acc(i, _):
                for k in range(DH // 16):
                    ksl = pl.ds(16 * k, 16)
                    abuf[i, ksl] = abuf[i, ksl] + gbuf[i, ksl]
                return 0

            lax.fori_loop(0, K, _acc, 0)
            pltpu.sync_copy(abuf, out2.at[c, sl])
        plsc.subcore_barrier()


@functools.partial(
    pl.kernel,
    out_type=jax.ShapeDtypeStruct((NC, NP, DH), jnp.float32),
    mesh=plsc.VectorSubcoreMesh(core_axis_name="c", subcore_axis_name="s"),
    scratch_types=[
        pltpu.VMEM_SHARED((NP, DH), jnp.float32),
        pltpu.VMEM_SHARED((NP, DH), jnp.float32),
        pltpu.VMEM((2, K), jnp.int32),
        pltpu.VMEM((1, K), jnp.float32),
        pltpu.VMEM((K, DH), jnp.float32),
        pltpu.VMEM((K, DH), jnp.float32),
    ],
)
def _graph_conv_sc(emb2, er, wr4, out2, *scratch):
    _sc_body(emb2, er, wr4, out2, *scratch)


def kernel(user_emb, entity_emb, graph_indices, graph_values):
    all_embed = jnp.concatenate([user_emb, entity_emb], axis=0)
    all_embed = jnp.pad(all_embed, ((0, NP - N), (0, 0)))
    emb2 = jnp.stack([all_embed[:, :DH], all_embed[:, DH:]], axis=0)
    head = graph_indices[0]
    tail = graph_indices[1]
    pad = E_PAD - E
    tailr = jnp.pad(tail, (0, pad)).reshape(NS, NCH, K)
    headr = jnp.pad(head, (0, pad)).reshape(NS, NCH, K)
    wr = jnp.pad(graph_values, (0, pad)).reshape(NS, NCH, K)
    er = jnp.stack([tailr, headr], axis=2)
    wr4 = wr[:, :, None, :]
    out2 = _graph_conv_sc(emb2, er, wr4)
    acc = jnp.concatenate([out2[0, :N], out2[1, :N]], axis=1)
    return (acc[:N_USERS], acc[N_USERS:])


# speedup vs baseline: 7.0460x; 1.6528x over previous
import functools

import jax
import jax.numpy as jnp
from jax import lax
from jax.experimental import pallas as pl
from jax.experimental.pallas import tpu as pltpu
from jax.experimental.pallas import tpu_sc as plsc

N_USERS = 2000
N = 10000
NP = 10240
D = 128
E = 320000
N_HOPS = 3

NC = 2
NS = 16
DH = D // NC
RPT = NP // NS
K = 128
GC = 8
NG = 20
NCH = NG * GC
EPT = NCH * K
E_PAD = NS * EPT
NQ = RPT // K


def _splat(i):
    return jnp.full((16,), i, dtype=jnp.int32)


_GDN = lax.GatherDimensionNumbers(
    offset_dims=(), collapsed_slice_dims=(0,), start_index_map=(0,))


def _bcast_lane(v16, lane):
    return lax.gather(v16, _splat(lane)[:, None], _GDN, (1,),
                      mode=lax.GatherScatterMode.PROMISE_IN_BOUNDS)


def _sc_body(emb2, er, wr5, out2, s0, s1, ebuf, wbuf, gbuf,
             se, sg, ss):
    c = lax.axis_index("c")
    s = lax.axis_index("s")
    row0 = s * RPT

    def fetch_group(g, slot):
        pltpu.async_copy(er.at[s, g], ebuf.at[slot], se.at[slot])
        pltpu.async_copy(wr5.at[s, g], wbuf.at[slot], se.at[slot])

    def wait_fetch(slot):
        pltpu.make_async_copy(er.at[s, 0], ebuf.at[slot], se.at[slot]).wait()
        pltpu.make_async_copy(wr5.at[s, 0], wbuf.at[slot], se.at[slot]).wait()

    def gather(cur, slot, k, b):
        pltpu.async_copy(cur.at[ebuf.at[slot, 2 * k]], gbuf.at[b], sg.at[b])

    def wait_gather(cur, slot, k, b):
        pltpu.make_async_copy(cur.at[ebuf.at[slot, 2 * k]], gbuf.at[b],
                              sg.at[b]).wait()

    def scatter(nxt, slot, k, b):
        pltpu.async_copy(gbuf.at[b], nxt.at[ebuf.at[slot, 2 * k + 1]],
                         ss.at[b], add=True)

    def drain_scatter(nxt, slot, k, b):
        pltpu.make_async_copy(gbuf.at[b], nxt.at[ebuf.at[slot, 2 * k + 1]],
                              ss.at[b]).wait()

    def scale(slot, k, b):
        def _scale16(g, _):
            w16 = wbuf[slot, k, pl.ds(16 * g, 16)]
            base = 16 * g
            for e16 in range(16):
                wbc = _bcast_lane(w16, e16)
                for q in range(DH // 16):
                    sl = pl.ds(16 * q, 16)
                    gbuf[b, base + e16, sl] = gbuf[b, base + e16, sl] * wbc
            return 0

        lax.fori_loop(0, K // 16, _scale16, 0)

    for q in range(NQ):
        sl = pl.ds(row0 + K * q, K)
        pltpu.sync_copy(emb2.at[c, sl], gbuf.at[0])
        pltpu.sync_copy(gbuf.at[0], s0.at[sl])
    plsc.subcore_barrier()

    for hop in range(N_HOPS):
        cur = s0 if hop % 2 == 0 else s1
        nxt = s1 if hop % 2 == 0 else s0

        def _zrow(i, _):
            for q in range(DH // 16):
                gbuf[0, i, pl.ds(16 * q, 16)] = jnp.zeros((16,), jnp.float32)
            return 0

        lax.fori_loop(0, K, _zrow, 0)
        for q in range(NQ):
            pltpu.sync_copy(gbuf.at[0], nxt.at[pl.ds(row0 + K * q, K)])
        plsc.subcore_barrier()

        def process_group(g, slot):
            wait_fetch(slot)

            @pl.when(g > 0)
            def _():
                drain_scatter(nxt, slot, 0, 0)
            gather(cur, slot, 0, 0)
            for k in range(GC):
                b = k % 2
                if k < GC - 1:
                    bn = (k + 1) % 2
                    if k == 0:
                        @pl.when(g > 0)
                        def _():
                            drain_scatter(nxt, slot, 1, bn)
                    else:
                        drain_scatter(nxt, slot, k - 1, bn)
                    gather(cur, slot, k + 1, bn)
                    if k == 1:
                        @pl.when(g < NG - 1)
                        def _():
                            fetch_group(g + 1, 1 - slot)
                wait_gather(cur, slot, k, b)
                scale(slot, k, b)
                scatter(nxt, slot, k, b)

        fetch_group(0, 0)

        @pl.loop(0, NG, step=2)
        def _pair(g):
            process_group(g, 0)
            process_group(g + 1, 1)

        drain_scatter(nxt, 1, GC - 2, 0)
        drain_scatter(nxt, 1, GC - 1, 1)
        plsc.subcore_barrier()

        for q in range(NQ):
            sl = pl.ds(row0 + K * q, K)
            pltpu.sync_copy(nxt.at[sl], gbuf.at[0])
            if hop == 0:
                pltpu.sync_copy(emb2.at[c, sl], gbuf.at[1])
            else:
                pltpu.sync_copy(out2.at[c, sl], gbuf.at[1])

            def _acc(i, _):
                for q2 in range(DH // 16):
                    ksl = pl.ds(16 * q2, 16)
                    gbuf[1, i, ksl] = gbuf[1, i, ksl] + gbuf[0, i, ksl]
                return 0

            lax.fori_loop(0, K, _acc, 0)
            pltpu.sync_copy(gbuf.at[1], out2.at[c, sl])
        plsc.subcore_barrier()


@functools.partial(
    pl.kernel,
    out_type=jax.ShapeDtypeStruct((NC, NP, DH), jnp.float32),
    mesh=plsc.VectorSubcoreMesh(core_axis_name="c", subcore_axis_name="s"),
    scratch_types=[
        pltpu.VMEM_SHARED((NP, DH), jnp.float32),
        pltpu.VMEM_SHARED((NP, DH), jnp.float32),
        pltpu.VMEM((2, 2 * GC, K), jnp.int32),
        pltpu.VMEM((2, GC, K), jnp.float32),
        pltpu.VMEM((2, K, DH), jnp.float32),
        pltpu.SemaphoreType.DMA((2,)),
        pltpu.SemaphoreType.DMA((2,)),
        pltpu.SemaphoreType.DMA((2,)),
    ],
)
def _graph_conv_sc(emb2, er, wr5, out2, *scratch):
    _sc_body(emb2, er, wr5, out2, *scratch)


def kernel(user_emb, entity_emb, graph_indices, graph_values):
    all_embed = jnp.concatenate([user_emb, entity_emb], axis=0)
    all_embed = jnp.pad(all_embed, ((0, NP - N), (0, 0)))
    emb2 = jnp.stack([all_embed[:, :DH], all_embed[:, DH:]], axis=0)
    head = graph_indices[0]
    tail = graph_indices[1]
    pad = E_PAD - E
    tailr = jnp.pad(tail, (0, pad)).reshape(NS, NG, GC, K)
    headr = jnp.pad(head, (0, pad)).reshape(NS, NG, GC, K)
    wr = jnp.pad(graph_values, (0, pad)).reshape(NS, NG, GC, K)
    er = jnp.stack([tailr, headr], axis=3).reshape(NS, NG, 2 * GC, K)
    out2 = _graph_conv_sc(emb2, er, wr)
    acc = jnp.concatenate([out2[0, :N], out2[1, :N]], axis=1)
    return (acc[:N_USERS], acc[N_USERS:])


# speedup vs baseline: 7.1561x; 1.0156x over previous
import functools

import jax
import jax.numpy as jnp
from jax import lax
from jax.experimental import pallas as pl
from jax.experimental.pallas import tpu as pltpu
from jax.experimental.pallas import tpu_sc as plsc

N_USERS = 2000
N = 10000
NP = 10240
D = 128
E = 320000
N_HOPS = 3

NC = 2
NS = 16
DH = D // NC
RPT = NP // NS
K = 128
GC = 8
NG = 20
NCH = NG * GC
EPT = NCH * K
E_PAD = NS * EPT
NQ = RPT // K


def _splat(i):
    return jnp.full((16,), i, dtype=jnp.int32)


_GDN = lax.GatherDimensionNumbers(
    offset_dims=(), collapsed_slice_dims=(0,), start_index_map=(0,))


def _bcast_lane(v16, lane):
    return lax.gather(v16, _splat(lane)[:, None], _GDN, (1,),
                      mode=lax.GatherScatterMode.PROMISE_IN_BOUNDS)


def _sc_body(emb2, er, wr5, out2, s0, s1, ebuf, wbuf, gbuf,
             se, sg, ss):
    c = lax.axis_index("c")
    s = lax.axis_index("s")
    row0 = s * RPT

    def fetch_group(g, slot):
        pltpu.async_copy(er.at[s, g], ebuf.at[slot], se.at[slot])
        pltpu.async_copy(wr5.at[s, g], wbuf.at[slot], se.at[slot])

    def wait_fetch(slot):
        pltpu.make_async_copy(er.at[s, 0], ebuf.at[slot], se.at[slot]).wait()
        pltpu.make_async_copy(wr5.at[s, 0], wbuf.at[slot], se.at[slot]).wait()

    def gather(cur, slot, k, b):
        pltpu.async_copy(cur.at[ebuf.at[slot, 2 * k]], gbuf.at[b], sg.at[b])

    def wait_gather(cur, slot, k, b):
        pltpu.make_async_copy(cur.at[ebuf.at[slot, 2 * k]], gbuf.at[b],
                              sg.at[b]).wait()

    def scatter(nxt, slot, k, b):
        pltpu.async_copy(gbuf.at[b], nxt.at[ebuf.at[slot, 2 * k + 1]],
                         ss.at[b], add=True)

    def drain_scatter(nxt, slot, k, b):
        pltpu.make_async_copy(gbuf.at[b], nxt.at[ebuf.at[slot, 2 * k + 1]],
                              ss.at[b]).wait()

    def scale(slot, k, b):
        def _scale32(g, _):
            for h in range(2):
                w16 = wbuf[slot, k, pl.ds(32 * g + 16 * h, 16)]
                base = 32 * g + 16 * h
                for e16 in range(16):
                    wbc = _bcast_lane(w16, e16)
                    for q in range(DH // 16):
                        sl = pl.ds(16 * q, 16)
                        gbuf[b, base + e16, sl] = gbuf[b, base + e16, sl] * wbc
            return 0

        lax.fori_loop(0, K // 32, _scale32, 0)

    for q in range(NQ):
        sl = pl.ds(row0 + K * q, K)
        pltpu.sync_copy(emb2.at[c, sl], gbuf.at[0])
        pltpu.sync_copy(gbuf.at[0], s0.at[sl])
    plsc.subcore_barrier()

    for hop in range(N_HOPS):
        cur = s0 if hop % 2 == 0 else s1
        nxt = s1 if hop % 2 == 0 else s0

        def _zrow(i, _):
            for q in range(DH // 16):
                gbuf[0, i, pl.ds(16 * q, 16)] = jnp.zeros((16,), jnp.float32)
            return 0

        lax.fori_loop(0, K, _zrow, 0)
        for q in range(NQ):
            pltpu.sync_copy(gbuf.at[0], nxt.at[pl.ds(row0 + K * q, K)])
        plsc.subcore_barrier()

        def process_group(g, slot):
            wait_fetch(slot)

            @pl.when(g > 0)
            def _():
                drain_scatter(nxt, slot, 0, 0)
                drain_scatter(nxt, slot, 1, 1)

            @pl.when(g < NG - 1)
            def _():
                fetch_group(g + 1, 1 - slot)

            @pl.loop(0, GC, step=2)
            def _chunkpair(k):
                @pl.when(k > 0)
                def _():
                    drain_scatter(nxt, slot, 0, 0)
                gather(cur, slot, k, 0)

                @pl.when(k > 0)
                def _():
                    drain_scatter(nxt, slot, 1, 1)
                gather(cur, slot, k + 1, 1)
                wait_gather(cur, slot, k, 0)
                scale(slot, k, 0)
                scatter(nxt, slot, k, 0)
                wait_gather(cur, slot, k + 1, 1)
                scale(slot, k + 1, 1)
                scatter(nxt, slot, k + 1, 1)

        fetch_group(0, 0)

        @pl.loop(0, NG, step=2)
        def _pair(g):
            process_group(g, 0)
            process_group(g + 1, 1)

        drain_scatter(nxt, 1, GC - 2, 0)
        drain_scatter(nxt, 1, GC - 1, 1)
        plsc.subcore_barrier()

        for q in range(NQ):
            sl = pl.ds(row0 + K * q, K)
            pltpu.sync_copy(nxt.at[sl], gbuf.at[0])
            if hop == 0:
                pltpu.sync_copy(emb2.at[c, sl], gbuf.at[1])
            else:
                pltpu.sync_copy(out2.at[c, sl], gbuf.at[1])

            def _acc(i, _):
                for q2 in range(DH // 16):
                    ksl = pl.ds(16 * q2, 16)
                    gbuf[1, i, ksl] = gbuf[1, i, ksl] + gbuf[0, i, ksl]
                return 0

            lax.fori_loop(0, K, _acc, 0)
            pltpu.sync_copy(gbuf.at[1], out2.at[c, sl])
        plsc.subcore_barrier()


@functools.partial(
    pl.kernel,
    out_type=jax.ShapeDtypeStruct((NC, NP, DH), jnp.float32),
    mesh=plsc.VectorSubcoreMesh(core_axis_name="c", subcore_axis_name="s"),
    scratch_types=[
        pltpu.VMEM_SHARED((NP, DH), jnp.float32),
        pltpu.VMEM_SHARED((NP, DH), jnp.float32),
        pltpu.VMEM((2, 2 * GC, K), jnp.int32),
        pltpu.VMEM((2, GC, K), jnp.float32),
        pltpu.VMEM((2, K, DH), jnp.float32),
        pltpu.SemaphoreType.DMA((2,)),
        pltpu.SemaphoreType.DMA((2,)),
        pltpu.SemaphoreType.DMA((2,)),
    ],
)
def _graph_conv_sc(emb2, er, wr5, out2, *scratch):
    _sc_body(emb2, er, wr5, out2, *scratch)


def kernel(user_emb, entity_emb, graph_indices, graph_values):
    all_embed = jnp.concatenate([user_emb, entity_emb], axis=0)
    all_embed = jnp.pad(all_embed, ((0, NP - N), (0, 0)))
    emb2 = jnp.stack([all_embed[:, :DH], all_embed[:, DH:]], axis=0)
    head = graph_indices[0]
    tail = graph_indices[1]
    pad = E_PAD - E
    tailr = jnp.pad(tail, (0, pad)).reshape(NS, NG, GC, K)
    headr = jnp.pad(head, (0, pad)).reshape(NS, NG, GC, K)
    wr = jnp.pad(graph_values, (0, pad)).reshape(NS, NG, GC, K)
    er = jnp.stack([tailr, headr], axis=3).reshape(NS, NG, 2 * GC, K)
    out2 = _graph_conv_sc(emb2, er, wr)
    acc = jnp.concatenate([out2[0, :N], out2[1, :N]], axis=1)
    return (acc[:N_USERS], acc[N_USERS:])
